# Initial kernel scaffold; baseline (speedup 1.0000x reference)
import jax
import jax.numpy as jnp
from jax.experimental import pallas as pl


def kernel(x, edge_index, batch, weight, W_ih, W_hh, b_ih, b_hh, lin_W, lin_b):
    raise NotImplementedError("write your pallas kernel here")


# speedup vs baseline: 4.1383x; 4.1383x over previous
import functools

import jax
import jax.numpy as jnp
from jax import lax
from jax.experimental import pallas as pl
from jax.experimental.pallas import tpu as pltpu
from jax.experimental.pallas import tpu_sc as plsc

N_NODES = 10000
N_EDGES = 320000
IN_CH = 128
HID = 256
NUM_CLASSES = 10
NUM_LAYERS = 3
NUM_GRAPHS = 64

HALF = HID // 2
N_SUBCORES = 16
EDGES_PER_TILE = N_EDGES // N_SUBCORES
CHUNK = 128
NFULL = EDGES_PER_TILE // CHUNK
TAIL = EDGES_PER_TILE - NFULL * CHUNK
ROWS_PER_TILE = 624
ROWS_EXTRA = N_NODES - N_SUBCORES * ROWS_PER_TILE


def _sc_seg_body(m_hbm, src_hbm, dst_hbm, out_hbm,
                 src_v, dst_v, rows_v, src_t, dst_t, rows_t, acc_sh, sem):
    c = lax.axis_index("c")
    s = lax.axis_index("s")

    zv = jnp.zeros((16,), jnp.float32)

    def zrow(r, carry):
        for k in range(HALF // 16):
            rows_v[r, k * 16:(k + 1) * 16] = zv
        return carry

    lax.fori_loop(0, CHUNK, zrow, 0)

    base_r = s * ROWS_PER_TILE
    nfull_r = ROWS_PER_TILE // CHUNK
    rem_r = ROWS_PER_TILE - nfull_r * CHUNK
    for k in range(nfull_r):
        pltpu.sync_copy(rows_v, acc_sh.at[pl.ds(base_r + k * CHUNK, CHUNK)])
    pltpu.sync_copy(rows_v.at[pl.ds(0, rem_r)],
                    acc_sh.at[pl.ds(base_r + nfull_r * CHUNK, rem_r)])

    @pl.when(s == 0)
    def _zero_extra():
        pltpu.sync_copy(rows_v.at[pl.ds(0, ROWS_EXTRA)],
                        acc_sh.at[pl.ds(N_SUBCORES * ROWS_PER_TILE, ROWS_EXTRA)])

    plsc.subcore_barrier()

    ebase = s * EDGES_PER_TILE

    def chunk_body(j, carry):
        off = ebase + j * CHUNK
        pltpu.sync_copy(src_hbm.at[pl.ds(off, CHUNK)], src_v)
        pltpu.sync_copy(dst_hbm.at[pl.ds(off, CHUNK)], dst_v)
        pltpu.async_copy(m_hbm.at[c].at[src_v], rows_v, sem).wait()
        pltpu.sync_copy(rows_v, acc_sh.at[dst_v], add=True)
        return carry

    lax.fori_loop(0, NFULL, chunk_body, 0)

    toff = ebase + NFULL * CHUNK
    pltpu.sync_copy(src_hbm.at[pl.ds(toff, TAIL)], src_t)
    pltpu.sync_copy(dst_hbm.at[pl.ds(toff, TAIL)], dst_t)
    pltpu.async_copy(m_hbm.at[c].at[src_t], rows_t, sem).wait()
    pltpu.sync_copy(rows_t, acc_sh.at[dst_t], add=True)

    plsc.subcore_barrier()

    for k in range(nfull_r):
        r0 = base_r + k * CHUNK
        pltpu.sync_copy(acc_sh.at[pl.ds(r0, CHUNK)], rows_v)
        pltpu.sync_copy(rows_v, out_hbm.at[c].at[pl.ds(r0, CHUNK)])
    r0 = base_r + nfull_r * CHUNK
    pltpu.sync_copy(acc_sh.at[pl.ds(r0, rem_r)], rows_v.at[pl.ds(0, rem_r)])
    pltpu.sync_copy(rows_v.at[pl.ds(0, rem_r)], out_hbm.at[c].at[pl.ds(r0, rem_r)])

    @pl.when(s == 0)
    def _write_extra():
        r1 = N_SUBCORES * ROWS_PER_TILE
        pltpu.sync_copy(acc_sh.at[pl.ds(r1, ROWS_EXTRA)],
                        rows_v.at[pl.ds(0, ROWS_EXTRA)])
        pltpu.sync_copy(rows_v.at[pl.ds(0, ROWS_EXTRA)],
                        out_hbm.at[c].at[pl.ds(r1, ROWS_EXTRA)])


_sc_segment_sum = functools.partial(
    pl.kernel,
    mesh=plsc.VectorSubcoreMesh(core_axis_name="c", subcore_axis_name="s"),
    out_type=jax.ShapeDtypeStruct((2, N_NODES, HALF), jnp.float32),
    scratch_types=[
        pltpu.VMEM((CHUNK,), jnp.int32),
        pltpu.VMEM((CHUNK,), jnp.int32),
        pltpu.VMEM((CHUNK, HALF), jnp.float32),
        pltpu.VMEM((TAIL,), jnp.int32),
        pltpu.VMEM((TAIL,), jnp.int32),
        pltpu.VMEM((TAIL, HALF), jnp.float32),
        pltpu.VMEM_SHARED((N_NODES, HALF), jnp.float32),
        pltpu.SemaphoreType.DMA,
    ],
)(_sc_seg_body)


def _mm_body(h_ref, w_ref, o_ref):
    o_ref[0] = jnp.dot(h_ref[...], w_ref[...],
                       preferred_element_type=jnp.float32)


def _matmul_split(h, w):
    bn = 2000
    return pl.pallas_call(
        _mm_body,
        grid=(N_NODES // bn, 2),
        in_specs=[
            pl.BlockSpec((bn, HID), lambda i, c: (i, 0)),
            pl.BlockSpec((HID, HALF), lambda i, c: (0, c)),
        ],
        out_specs=pl.BlockSpec((1, bn, HALF), lambda i, c: (c, i, 0)),
        out_shape=jax.ShapeDtypeStruct((2, N_NODES, HALF), jnp.float32),
    )(h, w)


def _gru_body(agg_ref, h_ref, wih_ref, whh_ref, bih_ref, bhh_ref, o_ref):
    agg = jnp.concatenate([agg_ref[0], agg_ref[1]], axis=1)
    h = h_ref[...]
    gi = lax.dot_general(agg, wih_ref[...], (((1,), (1,)), ((), ())),
                         preferred_element_type=jnp.float32) + bih_ref[...]
    gh = lax.dot_general(h, whh_ref[...], (((1,), (1,)), ((), ())),
                         preferred_element_type=jnp.float32) + bhh_ref[...]
    r = jax.nn.sigmoid(gi[:, :HID] + gh[:, :HID])
    z = jax.nn.sigmoid(gi[:, HID:2 * HID] + gh[:, HID:2 * HID])
    n = jnp.tanh(gi[:, 2 * HID:] + r * gh[:, 2 * HID:])
    o_ref[...] = (1.0 - z) * n + z * h


def _gru(agg2, h, W_ih, W_hh, bih2, bhh2):
    bn = 2000
    return pl.pallas_call(
        _gru_body,
        grid=(N_NODES // bn,),
        in_specs=[
            pl.BlockSpec((2, bn, HALF), lambda i: (0, i, 0)),
            pl.BlockSpec((bn, HID), lambda i: (i, 0)),
            pl.BlockSpec((3 * HID, HID), lambda i: (0, 0)),
            pl.BlockSpec((3 * HID, HID), lambda i: (0, 0)),
            pl.BlockSpec((1, 3 * HID), lambda i: (0, 0)),
            pl.BlockSpec((1, 3 * HID), lambda i: (0, 0)),
        ],
        out_specs=pl.BlockSpec((bn, HID), lambda i: (i, 0)),
        out_shape=jax.ShapeDtypeStruct((N_NODES, HID), jnp.float32),
    )(agg2, h, W_ih, W_hh, bih2, bhh2)


def _pool_body(h_ref, b_ref, lw_ref, lb_ref, o_ref):
    bvec = b_ref[0, :]
    gids = lax.broadcasted_iota(jnp.int32, (NUM_GRAPHS, N_NODES), 0)
    onehot = (bvec[None, :] == gids).astype(jnp.float32)
    sums = jnp.dot(onehot, h_ref[...], preferred_element_type=jnp.float32)
    counts = jnp.sum(onehot, axis=1)
    pooled = sums / jnp.clip(counts, 1.0)[:, None]
    o_ref[...] = lax.dot_general(pooled, lw_ref[...], (((1,), (1,)), ((), ())),
                                 preferred_element_type=jnp.float32) + lb_ref[...]


def _pool_classify(h, batch2, lwp, lbp):
    return pl.pallas_call(
        _pool_body,
        out_shape=jax.ShapeDtypeStruct((NUM_GRAPHS, 128), jnp.float32),
    )(h, batch2, lwp, lbp)


def kernel(x, edge_index, batch, weight, W_ih, W_hh, b_ih, b_hh, lin_W, lin_b):
    src = edge_index[0].astype(jnp.int32)
    dst = edge_index[1].astype(jnp.int32)
    h = jnp.pad(x, ((0, 0), (0, HID - x.shape[1])))
    bih2 = b_ih[None, :]
    bhh2 = b_hh[None, :]
    for i in range(NUM_LAYERS):
        m2 = _matmul_split(h, weight[i])
        agg2 = _sc_segment_sum(m2, src, dst)
        h = _gru(agg2, h, W_ih, W_hh, bih2, bhh2)
    lwp = jnp.zeros((128, HID), jnp.float32).at[:NUM_CLASSES].set(lin_W)
    lbp = jnp.zeros((1, 128), jnp.float32).at[0, :NUM_CLASSES].set(lin_b)
    out = _pool_classify(h, batch.astype(jnp.int32)[None, :], lwp, lbp)
    return out[:, :NUM_CLASSES]


# speedup vs baseline: 7.2813x; 1.7595x over previous
import functools

import jax
import jax.numpy as jnp
from jax import lax
from jax.experimental import pallas as pl
from jax.experimental.pallas import tpu as pltpu
from jax.experimental.pallas import tpu_sc as plsc

N_NODES = 10000
N_EDGES = 320000
IN_CH = 128
HID = 256
NUM_CLASSES = 10
NUM_LAYERS = 3
NUM_GRAPHS = 64

HALF = HID // 2
N_SUBCORES = 16
EDGES_PER_TILE = N_EDGES // N_SUBCORES
CHUNK = 128
NFULL = EDGES_PER_TILE // CHUNK
TAIL = EDGES_PER_TILE - NFULL * CHUNK
NSEC = 6
SEC_CHUNKS = NFULL // NSEC
ROWS_PER_TILE = 624
ROWS_EXTRA = N_NODES - N_SUBCORES * ROWS_PER_TILE


def _sc_seg_body(m_hbm, src_hbm, dst_hbm, out_hbm,
                 src_all, dst_all, dst_v0, dst_v1, rows_v, rows_v1,
                 src_t, dst_t, rows_t, acc_sh, sem):
    c = lax.axis_index("c")
    s = lax.axis_index("s")
    dst_bufs = (dst_v0, dst_v1)
    row_bufs = (rows_v, rows_v1)

    zv = jnp.zeros((16,), jnp.float32)

    def zrow(r, carry):
        for k in range(HALF // 16):
            rows_v[r, k * 16:(k + 1) * 16] = zv
        return carry

    lax.fori_loop(0, CHUNK, zrow, 0)

    base_r = s * ROWS_PER_TILE
    nfull_r = ROWS_PER_TILE // CHUNK
    rem_r = ROWS_PER_TILE - nfull_r * CHUNK
    for k in range(nfull_r):
        pltpu.sync_copy(rows_v, acc_sh.at[pl.ds(base_r + k * CHUNK, CHUNK)])
    pltpu.sync_copy(rows_v.at[pl.ds(0, rem_r)],
                    acc_sh.at[pl.ds(base_r + nfull_r * CHUNK, rem_r)])

    @pl.when(s == 0)
    def _zero_extra():
        pltpu.sync_copy(rows_v.at[pl.ds(0, ROWS_EXTRA)],
                        acc_sh.at[pl.ds(N_SUBCORES * ROWS_PER_TILE, ROWS_EXTRA)])

    plsc.subcore_barrier()

    ebase = s * EDGES_PER_TILE

    def stage_dst(j, buf):
        for k in range(CHUNK // 16):
            buf[k * 16:(k + 1) * 16] = dst_all[pl.ds(j * CHUNK + k * 16, 16)]

    for sec in range(NSEC):
        soff = ebase + sec * SEC_CHUNKS * CHUNK
        pltpu.sync_copy(src_hbm.at[pl.ds(soff, SEC_CHUNKS * CHUNK)], src_all)
        pltpu.sync_copy(dst_hbm.at[pl.ds(soff, SEC_CHUNKS * CHUNK)], dst_all)

        def pair_body(t, carry):
            for phase in range(2):
                j = 2 * t + phase
                cp = pltpu.async_copy(
                    m_hbm.at[c].at[src_all.at[pl.ds(j * CHUNK, CHUNK)]],
                    row_bufs[phase], sem)
                if phase == 0:
                    if sec == 0:
                        @pl.when(t > 0)
                        def _scatter_prev():
                            pltpu.sync_copy(row_bufs[1],
                                            acc_sh.at[dst_bufs[1]], add=True)
                    else:
                        pltpu.sync_copy(row_bufs[1], acc_sh.at[dst_bufs[1]],
                                        add=True)
                else:
                    pltpu.sync_copy(row_bufs[0], acc_sh.at[dst_bufs[0]],
                                    add=True)
                stage_dst(j, dst_bufs[phase])
                cp.wait()
            return carry

        lax.fori_loop(0, SEC_CHUNKS // 2, pair_body, 0)

    pltpu.sync_copy(row_bufs[1], acc_sh.at[dst_bufs[1]], add=True)

    toff = ebase + NFULL * CHUNK
    pltpu.sync_copy(src_hbm.at[pl.ds(toff, TAIL)], src_t)
    pltpu.sync_copy(dst_hbm.at[pl.ds(toff, TAIL)], dst_t)
    pltpu.async_copy(m_hbm.at[c].at[src_t], rows_t, sem).wait()
    pltpu.sync_copy(rows_t, acc_sh.at[dst_t], add=True)

    plsc.subcore_barrier()

    for k in range(nfull_r):
        r0 = base_r + k * CHUNK
        pltpu.sync_copy(acc_sh.at[pl.ds(r0, CHUNK)], rows_v)
        pltpu.sync_copy(rows_v, out_hbm.at[c].at[pl.ds(r0, CHUNK)])
    r0 = base_r + nfull_r * CHUNK
    pltpu.sync_copy(acc_sh.at[pl.ds(r0, rem_r)], rows_v.at[pl.ds(0, rem_r)])
    pltpu.sync_copy(rows_v.at[pl.ds(0, rem_r)], out_hbm.at[c].at[pl.ds(r0, rem_r)])

    @pl.when(s == 0)
    def _write_extra():
        r1 = N_SUBCORES * ROWS_PER_TILE
        pltpu.sync_copy(acc_sh.at[pl.ds(r1, ROWS_EXTRA)],
                        rows_v.at[pl.ds(0, ROWS_EXTRA)])
        pltpu.sync_copy(rows_v.at[pl.ds(0, ROWS_EXTRA)],
                        out_hbm.at[c].at[pl.ds(r1, ROWS_EXTRA)])


_sc_segment_sum = functools.partial(
    pl.kernel,
    mesh=plsc.VectorSubcoreMesh(core_axis_name="c", subcore_axis_name="s"),
    out_type=jax.ShapeDtypeStruct((2, N_NODES, HALF), jnp.float32),
    scratch_types=[
        pltpu.VMEM((SEC_CHUNKS * CHUNK,), jnp.int32),
        pltpu.VMEM((SEC_CHUNKS * CHUNK,), jnp.int32),
        pltpu.VMEM((CHUNK,), jnp.int32),
        pltpu.VMEM((CHUNK,), jnp.int32),
        pltpu.VMEM((CHUNK, HALF), jnp.float32),
        pltpu.VMEM((CHUNK, HALF), jnp.float32),
        pltpu.VMEM((TAIL,), jnp.int32),
        pltpu.VMEM((TAIL,), jnp.int32),
        pltpu.VMEM((TAIL, HALF), jnp.float32),
        pltpu.VMEM_SHARED((N_NODES, HALF), jnp.float32),
        pltpu.SemaphoreType.DMA,
    ],
)(_sc_seg_body)


def _mm_body(h_ref, w_ref, o_ref):
    o_ref[0] = jnp.dot(h_ref[...], w_ref[...],
                       preferred_element_type=jnp.float32)


def _matmul_split(h, w):
    bn = 2000
    return pl.pallas_call(
        _mm_body,
        grid=(N_NODES // bn, 2),
        in_specs=[
            pl.BlockSpec((bn, HID), lambda i, c: (i, 0)),
            pl.BlockSpec((HID, HALF), lambda i, c: (0, c)),
        ],
        out_specs=pl.BlockSpec((1, bn, HALF), lambda i, c: (c, i, 0)),
        out_shape=jax.ShapeDtypeStruct((2, N_NODES, HALF), jnp.float32),
    )(h, w)


def _gru_body(agg_ref, h_ref, wih_ref, whh_ref, bih_ref, bhh_ref, o_ref):
    agg = jnp.concatenate([agg_ref[0], agg_ref[1]], axis=1)
    h = h_ref[...]
    gi = lax.dot_general(agg, wih_ref[...], (((1,), (1,)), ((), ())),
                         preferred_element_type=jnp.float32) + bih_ref[...]
    gh = lax.dot_general(h, whh_ref[...], (((1,), (1,)), ((), ())),
                         preferred_element_type=jnp.float32) + bhh_ref[...]
    r = jax.nn.sigmoid(gi[:, :HID] + gh[:, :HID])
    z = jax.nn.sigmoid(gi[:, HID:2 * HID] + gh[:, HID:2 * HID])
    n = jnp.tanh(gi[:, 2 * HID:] + r * gh[:, 2 * HID:])
    o_ref[...] = (1.0 - z) * n + z * h


def _gru(agg2, h, W_ih, W_hh, bih2, bhh2):
    bn = 2000
    return pl.pallas_call(
        _gru_body,
        grid=(N_NODES // bn,),
        in_specs=[
            pl.BlockSpec((2, bn, HALF), lambda i: (0, i, 0)),
            pl.BlockSpec((bn, HID), lambda i: (i, 0)),
            pl.BlockSpec((3 * HID, HID), lambda i: (0, 0)),
            pl.BlockSpec((3 * HID, HID), lambda i: (0, 0)),
            pl.BlockSpec((1, 3 * HID), lambda i: (0, 0)),
            pl.BlockSpec((1, 3 * HID), lambda i: (0, 0)),
        ],
        out_specs=pl.BlockSpec((bn, HID), lambda i: (i, 0)),
        out_shape=jax.ShapeDtypeStruct((N_NODES, HID), jnp.float32),
    )(agg2, h, W_ih, W_hh, bih2, bhh2)


def _pool_body(h_ref, b_ref, lw_ref, lb_ref, o_ref):
    bvec = b_ref[0, :]
    gids = lax.broadcasted_iota(jnp.int32, (NUM_GRAPHS, N_NODES), 0)
    onehot = (bvec[None, :] == gids).astype(jnp.float32)
    sums = jnp.dot(onehot, h_ref[...], preferred_element_type=jnp.float32)
    counts = jnp.sum(onehot, axis=1)
    pooled = sums / jnp.clip(counts, 1.0)[:, None]
    o_ref[...] = lax.dot_general(pooled, lw_ref[...], (((1,), (1,)), ((), ())),
                                 preferred_element_type=jnp.float32) + lb_ref[...]


def _pool_classify(h, batch2, lwp, lbp):
    return pl.pallas_call(
        _pool_body,
        out_shape=jax.ShapeDtypeStruct((NUM_GRAPHS, 128), jnp.float32),
    )(h, batch2, lwp, lbp)


def kernel(x, edge_index, batch, weight, W_ih, W_hh, b_ih, b_hh, lin_W, lin_b):
    src = edge_index[0].astype(jnp.int32)
    dst = edge_index[1].astype(jnp.int32)
    h = jnp.pad(x, ((0, 0), (0, HID - x.shape[1])))
    bih2 = b_ih[None, :]
    bhh2 = b_hh[None, :]
    for i in range(NUM_LAYERS):
        m2 = _matmul_split(h, weight[i])
        agg2 = _sc_segment_sum(m2, src, dst)
        h = _gru(agg2, h, W_ih, W_hh, bih2, bhh2)
    lwp = jnp.zeros((128, HID), jnp.float32).at[:NUM_CLASSES].set(lin_W)
    lbp = jnp.zeros((1, 128), jnp.float32).at[0, :NUM_CLASSES].set(lin_b)
    out = _pool_classify(h, batch.astype(jnp.int32)[None, :], lwp, lbp)
    return out[:, :NUM_CLASSES]


# speedup vs baseline: 7.4900x; 1.0287x over previous
import functools

import jax
import jax.numpy as jnp
from jax import lax
from jax.experimental import pallas as pl
from jax.experimental.pallas import tpu as pltpu
from jax.experimental.pallas import tpu_sc as plsc

N_NODES = 10000
N_EDGES = 320000
IN_CH = 128
HID = 256
NUM_CLASSES = 10
NUM_LAYERS = 3
NUM_GRAPHS = 64

HALF = HID // 2
N_SUBCORES = 16
EDGES_PER_TILE = N_EDGES // N_SUBCORES
CHUNK = 128
NFULL = EDGES_PER_TILE // CHUNK
TAIL = EDGES_PER_TILE - NFULL * CHUNK
NSEC = 6
SEC_CHUNKS = NFULL // NSEC
ROWS_PER_TILE = 624
ROWS_EXTRA = N_NODES - N_SUBCORES * ROWS_PER_TILE


def _sc_seg_body(m_hbm, src_hbm, dst_hbm, out_hbm,
                 src_all, dst_all, dst_v0, dst_v1, rows_v, rows_v1,
                 src_t, dst_t, rows_t, acc_sh, sem, ss0, ss1):
    c = lax.axis_index("c")
    s = lax.axis_index("s")
    dst_bufs = (dst_v0, dst_v1)
    row_bufs = (rows_v, rows_v1)
    sc_sems = (ss0, ss1)

    zv = jnp.zeros((16,), jnp.float32)

    def zrow(r, carry):
        for k in range(HALF // 16):
            rows_v[r, k * 16:(k + 1) * 16] = zv
        return carry

    lax.fori_loop(0, CHUNK, zrow, 0)

    base_r = s * ROWS_PER_TILE
    nfull_r = ROWS_PER_TILE // CHUNK
    rem_r = ROWS_PER_TILE - nfull_r * CHUNK
    for k in range(nfull_r):
        pltpu.sync_copy(rows_v, acc_sh.at[pl.ds(base_r + k * CHUNK, CHUNK)])
    pltpu.sync_copy(rows_v.at[pl.ds(0, rem_r)],
                    acc_sh.at[pl.ds(base_r + nfull_r * CHUNK, rem_r)])

    @pl.when(s == 0)
    def _zero_extra():
        pltpu.sync_copy(rows_v.at[pl.ds(0, ROWS_EXTRA)],
                        acc_sh.at[pl.ds(N_SUBCORES * ROWS_PER_TILE, ROWS_EXTRA)])

    plsc.subcore_barrier()

    ebase = s * EDGES_PER_TILE

    def stage_dst(j, buf):
        for k in range(CHUNK // 16):
            buf[k * 16:(k + 1) * 16] = dst_all[pl.ds(j * CHUNK + k * 16, 16)]

    for sec in range(NSEC):
        soff = ebase + sec * SEC_CHUNKS * CHUNK
        pltpu.sync_copy(src_hbm.at[pl.ds(soff, SEC_CHUNKS * CHUNK)], src_all)
        pltpu.sync_copy(dst_hbm.at[pl.ds(soff, SEC_CHUNKS * CHUNK)], dst_all)

        def pair_body(t, carry):
            for phase in range(2):
                j = 2 * t + phase
                if sec == 0:
                    @pl.when(t > 0)
                    def _wait_prev():
                        pltpu.make_async_copy(
                            row_bufs[phase], acc_sh.at[dst_bufs[phase]],
                            sc_sems[phase]).wait()
                else:
                    pltpu.make_async_copy(
                        row_bufs[phase], acc_sh.at[dst_bufs[phase]],
                        sc_sems[phase]).wait()
                cp = pltpu.async_copy(
                    m_hbm.at[c].at[src_all.at[pl.ds(j * CHUNK, CHUNK)]],
                    row_bufs[phase], sem)
                stage_dst(j, dst_bufs[phase])
                cp.wait()
                pltpu.async_copy(row_bufs[phase], acc_sh.at[dst_bufs[phase]],
                                 sc_sems[phase], add=True)
            return carry

        lax.fori_loop(0, SEC_CHUNKS // 2, pair_body, 0)

    for phase in range(2):
        pltpu.make_async_copy(row_bufs[phase], acc_sh.at[dst_bufs[phase]],
                              sc_sems[phase]).wait()

    toff = ebase + NFULL * CHUNK
    pltpu.sync_copy(src_hbm.at[pl.ds(toff, TAIL)], src_t)
    pltpu.sync_copy(dst_hbm.at[pl.ds(toff, TAIL)], dst_t)
    pltpu.async_copy(m_hbm.at[c].at[src_t], rows_t, sem).wait()
    pltpu.sync_copy(rows_t, acc_sh.at[dst_t], add=True)

    plsc.subcore_barrier()

    for k in range(nfull_r):
        r0 = base_r + k * CHUNK
        pltpu.sync_copy(acc_sh.at[pl.ds(r0, CHUNK)], rows_v)
        pltpu.sync_copy(rows_v, out_hbm.at[c].at[pl.ds(r0, CHUNK)])
    r0 = base_r + nfull_r * CHUNK
    pltpu.sync_copy(acc_sh.at[pl.ds(r0, rem_r)], rows_v.at[pl.ds(0, rem_r)])
    pltpu.sync_copy(rows_v.at[pl.ds(0, rem_r)], out_hbm.at[c].at[pl.ds(r0, rem_r)])

    @pl.when(s == 0)
    def _write_extra():
        r1 = N_SUBCORES * ROWS_PER_TILE
        pltpu.sync_copy(acc_sh.at[pl.ds(r1, ROWS_EXTRA)],
                        rows_v.at[pl.ds(0, ROWS_EXTRA)])
        pltpu.sync_copy(rows_v.at[pl.ds(0, ROWS_EXTRA)],
                        out_hbm.at[c].at[pl.ds(r1, ROWS_EXTRA)])


_sc_segment_sum = functools.partial(
    pl.kernel,
    mesh=plsc.VectorSubcoreMesh(core_axis_name="c", subcore_axis_name="s"),
    out_type=jax.ShapeDtypeStruct((2, N_NODES, HALF), jnp.float32),
    scratch_types=[
        pltpu.VMEM((SEC_CHUNKS * CHUNK,), jnp.int32),
        pltpu.VMEM((SEC_CHUNKS * CHUNK,), jnp.int32),
        pltpu.VMEM((CHUNK,), jnp.int32),
        pltpu.VMEM((CHUNK,), jnp.int32),
        pltpu.VMEM((CHUNK, HALF), jnp.float32),
        pltpu.VMEM((CHUNK, HALF), jnp.float32),
        pltpu.VMEM((TAIL,), jnp.int32),
        pltpu.VMEM((TAIL,), jnp.int32),
        pltpu.VMEM((TAIL, HALF), jnp.float32),
        pltpu.VMEM_SHARED((N_NODES, HALF), jnp.float32),
        pltpu.SemaphoreType.DMA,
        pltpu.SemaphoreType.DMA,
        pltpu.SemaphoreType.DMA,
    ],
)(_sc_seg_body)


def _mm_body(h_ref, w_ref, o_ref):
    o_ref[0] = jnp.dot(h_ref[...], w_ref[...],
                       preferred_element_type=jnp.float32)


def _matmul_split(h, w):
    bn = 2000
    return pl.pallas_call(
        _mm_body,
        grid=(N_NODES // bn, 2),
        in_specs=[
            pl.BlockSpec((bn, HID), lambda i, c: (i, 0)),
            pl.BlockSpec((HID, HALF), lambda i, c: (0, c)),
        ],
        out_specs=pl.BlockSpec((1, bn, HALF), lambda i, c: (c, i, 0)),
        out_shape=jax.ShapeDtypeStruct((2, N_NODES, HALF), jnp.float32),
    )(h, w)


def _gru_body(agg_ref, h_ref, wih_ref, whh_ref, bih_ref, bhh_ref, o_ref):
    agg = jnp.concatenate([agg_ref[0], agg_ref[1]], axis=1)
    h = h_ref[...]
    gi = lax.dot_general(agg, wih_ref[...], (((1,), (1,)), ((), ())),
                         preferred_element_type=jnp.float32) + bih_ref[...]
    gh = lax.dot_general(h, whh_ref[...], (((1,), (1,)), ((), ())),
                         preferred_element_type=jnp.float32) + bhh_ref[...]
    r = jax.nn.sigmoid(gi[:, :HID] + gh[:, :HID])
    z = jax.nn.sigmoid(gi[:, HID:2 * HID] + gh[:, HID:2 * HID])
    n = jnp.tanh(gi[:, 2 * HID:] + r * gh[:, 2 * HID:])
    o_ref[...] = (1.0 - z) * n + z * h


def _gru_mm_body(agg_ref, h_ref, wih_ref, whh_ref, bih_ref, bhh_ref,
                 wnext_ref, h_out, m_out):
    agg = jnp.concatenate([agg_ref[0], agg_ref[1]], axis=1)
    h = h_ref[...]
    gi = lax.dot_general(agg, wih_ref[...], (((1,), (1,)), ((), ())),
                         preferred_element_type=jnp.float32) + bih_ref[...]
    gh = lax.dot_general(h, whh_ref[...], (((1,), (1,)), ((), ())),
                         preferred_element_type=jnp.float32) + bhh_ref[...]
    r = jax.nn.sigmoid(gi[:, :HID] + gh[:, :HID])
    z = jax.nn.sigmoid(gi[:, HID:2 * HID] + gh[:, HID:2 * HID])
    n = jnp.tanh(gi[:, 2 * HID:] + r * gh[:, 2 * HID:])
    hn = (1.0 - z) * n + z * h
    h_out[...] = hn
    mm = jnp.dot(hn, wnext_ref[...], preferred_element_type=jnp.float32)
    m_out[0] = mm[:, :HALF]
    m_out[1] = mm[:, HALF:]


def _gru_mm(agg2, h, W_ih, W_hh, bih2, bhh2, wnext):
    bn = 2000
    return pl.pallas_call(
        _gru_mm_body,
        grid=(N_NODES // bn,),
        in_specs=[
            pl.BlockSpec((2, bn, HALF), lambda i: (0, i, 0)),
            pl.BlockSpec((bn, HID), lambda i: (i, 0)),
            pl.BlockSpec((3 * HID, HID), lambda i: (0, 0)),
            pl.BlockSpec((3 * HID, HID), lambda i: (0, 0)),
            pl.BlockSpec((1, 3 * HID), lambda i: (0, 0)),
            pl.BlockSpec((1, 3 * HID), lambda i: (0, 0)),
            pl.BlockSpec((HID, HID), lambda i: (0, 0)),
        ],
        out_specs=[
            pl.BlockSpec((bn, HID), lambda i: (i, 0)),
            pl.BlockSpec((2, bn, HALF), lambda i: (0, i, 0)),
        ],
        out_shape=[
            jax.ShapeDtypeStruct((N_NODES, HID), jnp.float32),
            jax.ShapeDtypeStruct((2, N_NODES, HALF), jnp.float32),
        ],
    )(agg2, h, W_ih, W_hh, bih2, bhh2, wnext)


def _gru(agg2, h, W_ih, W_hh, bih2, bhh2):
    bn = 2000
    return pl.pallas_call(
        _gru_body,
        grid=(N_NODES // bn,),
        in_specs=[
            pl.BlockSpec((2, bn, HALF), lambda i: (0, i, 0)),
            pl.BlockSpec((bn, HID), lambda i: (i, 0)),
            pl.BlockSpec((3 * HID, HID), lambda i: (0, 0)),
            pl.BlockSpec((3 * HID, HID), lambda i: (0, 0)),
            pl.BlockSpec((1, 3 * HID), lambda i: (0, 0)),
            pl.BlockSpec((1, 3 * HID), lambda i: (0, 0)),
        ],
        out_specs=pl.BlockSpec((bn, HID), lambda i: (i, 0)),
        out_shape=jax.ShapeDtypeStruct((N_NODES, HID), jnp.float32),
    )(agg2, h, W_ih, W_hh, bih2, bhh2)


def _pool_body(h_ref, b_ref, lw_ref, lb_ref, o_ref):
    bvec = b_ref[0, :]
    gids = lax.broadcasted_iota(jnp.int32, (NUM_GRAPHS, N_NODES), 0)
    onehot = (bvec[None, :] == gids).astype(jnp.float32)
    sums = jnp.dot(onehot, h_ref[...], preferred_element_type=jnp.float32)
    counts = jnp.sum(onehot, axis=1)
    pooled = sums / jnp.clip(counts, 1.0)[:, None]
    o_ref[...] = lax.dot_general(pooled, lw_ref[...], (((1,), (1,)), ((), ())),
                                 preferred_element_type=jnp.float32) + lb_ref[...]


def _pool_classify(h, batch2, lwp, lbp):
    return pl.pallas_call(
        _pool_body,
        out_shape=jax.ShapeDtypeStruct((NUM_GRAPHS, 128), jnp.float32),
    )(h, batch2, lwp, lbp)


def kernel(x, edge_index, batch, weight, W_ih, W_hh, b_ih, b_hh, lin_W, lin_b):
    src = edge_index[0].astype(jnp.int32)
    dst = edge_index[1].astype(jnp.int32)
    h = jnp.pad(x, ((0, 0), (0, HID - x.shape[1])))
    bih2 = b_ih[None, :]
    bhh2 = b_hh[None, :]
    m2 = _matmul_split(h, weight[0])
    for i in range(NUM_LAYERS - 1):
        agg2 = _sc_segment_sum(m2, src, dst)
        h, m2 = _gru_mm(agg2, h, W_ih, W_hh, bih2, bhh2, weight[i + 1])
    agg2 = _sc_segment_sum(m2, src, dst)
    h = _gru(agg2, h, W_ih, W_hh, bih2, bhh2)
    lwp = jnp.zeros((128, HID), jnp.float32).at[:NUM_CLASSES].set(lin_W)
    lbp = jnp.zeros((1, 128), jnp.float32).at[0, :NUM_CLASSES].set(lin_b)
    out = _pool_classify(h, batch.astype(jnp.int32)[None, :], lwp, lbp)
    return out[:, :NUM_CLASSES]


# speedup vs baseline: 9.5648x; 1.2770x over previous
import functools

import jax
import jax.numpy as jnp
from jax import lax
from jax.experimental import pallas as pl
from jax.experimental.pallas import tpu as pltpu
from jax.experimental.pallas import tpu_sc as plsc

N_NODES = 10000
N_EDGES = 320000
IN_CH = 128
HID = 256
NUM_CLASSES = 10
NUM_LAYERS = 3
NUM_GRAPHS = 64

HALF = HID // 2
N_SUBCORES = 16
EDGES_PER_TILE = N_EDGES // N_SUBCORES
CHUNK = 112
NFULL = EDGES_PER_TILE // CHUNK
TAIL = EDGES_PER_TILE - NFULL * CHUNK
SECL = 12
NSEC_FULL = 14
LEFT = NFULL - NSEC_FULL * SECL
ROWS_PER_TILE = 624
ROWS_EXTRA = N_NODES - N_SUBCORES * ROWS_PER_TILE


def _sc_seg_body(m_hbm, src_hbm, dst_hbm, out_hbm,
                 srcA, dstA, srcB, dstB,
                 dv0, dv1, dv2, r0, r1, r2,
                 src_t, dst_t, acc_sh,
                 sem, sg0, sg1, sg2, ss0, ss1, ss2, si0, si1):
    c = lax.axis_index("c")
    s = lax.axis_index("s")
    rows = (r0, r1, r2)
    dvs = (dv0, dv1, dv2)
    sgs = (sg0, sg1, sg2)
    sss = (ss0, ss1, ss2)
    idxbufs = ((srcA, dstA), (srcB, dstB))

    zv = jnp.zeros((16,), jnp.float32)

    def zrow(rr, carry):
        for k in range(HALF // 16):
            r0[rr, k * 16:(k + 1) * 16] = zv
        return carry

    lax.fori_loop(0, CHUNK, zrow, 0)

    base_r = s * ROWS_PER_TILE
    nfull_r = ROWS_PER_TILE // CHUNK
    rem_r = ROWS_PER_TILE - nfull_r * CHUNK
    for k in range(nfull_r):
        pltpu.sync_copy(r0, acc_sh.at[pl.ds(base_r + k * CHUNK, CHUNK)])
    pltpu.sync_copy(r0.at[pl.ds(0, rem_r)],
                    acc_sh.at[pl.ds(base_r + nfull_r * CHUNK, rem_r)])

    @pl.when(s == 0)
    def _zero_extra():
        pltpu.sync_copy(r0.at[pl.ds(0, ROWS_EXTRA)],
                        acc_sh.at[pl.ds(N_SUBCORES * ROWS_PER_TILE, ROWS_EXTRA)])

    plsc.subcore_barrier()

    ebase = s * EDGES_PER_TILE

    def wait_scatter(q):
        pltpu.make_async_copy(rows[q], acc_sh.at[dvs[q]], sss[q]).wait()

    def issue_gather(srcbuf, off, q):
        pltpu.async_copy(m_hbm.at[c].at[srcbuf.at[pl.ds(off, CHUNK)]],
                         rows[q], sgs[q])

    def wait_gather(q):
        pltpu.make_async_copy(m_hbm.at[c].at[dvs[q]], rows[q], sgs[q]).wait()

    def step(q, loc, dstbuf, do_wait_scatter, pre):
        qn = (q + 1) % 3
        if do_wait_scatter:
            wait_scatter(qn)
        if pre is not None:
            pbuf, poff = pre
            issue_gather(pbuf, poff, qn)
        wait_gather(q)
        for k in range(CHUNK // 16):
            dvs[q][k * 16:(k + 1) * 16] = dstbuf[pl.ds(loc * CHUNK + k * 16,
                                                       16)]
        pltpu.async_copy(rows[q], acc_sh.at[dvs[q]], sss[q], add=True)

    pltpu.sync_copy(src_hbm.at[pl.ds(ebase, SECL * CHUNK)], srcA)
    pltpu.sync_copy(dst_hbm.at[pl.ds(ebase, SECL * CHUNK)], dstA)
    issue_gather(srcA, 0, 0)

    for sec in range(NSEC_FULL + 1):
        cur_src, cur_dst = idxbufs[sec % 2]
        nxt_src, nxt_dst = idxbufs[(sec + 1) % 2]
        if sec < NSEC_FULL:
            nlen = SECL if sec < NSEC_FULL - 1 else LEFT
            noff = ebase + (sec + 1) * SECL * CHUNK
            icp_s = pltpu.async_copy(src_hbm.at[pl.ds(noff, nlen * CHUNK)],
                                     nxt_src.at[pl.ds(0, nlen * CHUNK)], si0)
            icp_d = pltpu.async_copy(dst_hbm.at[pl.ds(noff, nlen * CHUNK)],
                                     nxt_dst.at[pl.ds(0, nlen * CHUNK)], si1)

        def trip(t, carry):
            for ph in range(3):
                loc = 3 * t + ph
                pre = (cur_src, (loc + 1) * CHUNK)
                if sec == 0 and ph < 2:
                    qn = (ph + 1) % 3

                    @pl.when(t > 0)
                    def _w():
                        wait_scatter(qn)

                    step(ph, loc, cur_dst, False, pre)
                else:
                    step(ph, loc, cur_dst, True, pre)
            return carry

        lax.fori_loop(0, 3, trip, 0)

        if sec < NSEC_FULL:
            step(0, 9, cur_dst, True, (cur_src, 10 * CHUNK))
            step(1, 10, cur_dst, True, (cur_src, 11 * CHUNK))
            icp_s.wait()
            icp_d.wait()
            step(2, 11, cur_dst, True, (nxt_src, 0))
        else:
            step(0, 9, cur_dst, True, None)

    wait_scatter(2)
    wait_scatter(0)

    toff = ebase + NFULL * CHUNK
    pltpu.sync_copy(src_hbm.at[pl.ds(toff, TAIL)], src_t)
    pltpu.sync_copy(dst_hbm.at[pl.ds(toff, TAIL)], dst_t)
    pltpu.async_copy(m_hbm.at[c].at[src_t], r0.at[pl.ds(0, TAIL)], sem).wait()
    pltpu.sync_copy(r0.at[pl.ds(0, TAIL)], acc_sh.at[dst_t], add=True)

    plsc.subcore_barrier()

    for k in range(nfull_r):
        w0 = base_r + k * CHUNK
        pltpu.sync_copy(acc_sh.at[pl.ds(w0, CHUNK)], r0)
        pltpu.sync_copy(r0, out_hbm.at[c].at[pl.ds(w0, CHUNK)])
    w0 = base_r + nfull_r * CHUNK
    pltpu.sync_copy(acc_sh.at[pl.ds(w0, rem_r)], r0.at[pl.ds(0, rem_r)])
    pltpu.sync_copy(r0.at[pl.ds(0, rem_r)], out_hbm.at[c].at[pl.ds(w0, rem_r)])

    @pl.when(s == 0)
    def _write_extra():
        w1 = N_SUBCORES * ROWS_PER_TILE
        pltpu.sync_copy(acc_sh.at[pl.ds(w1, ROWS_EXTRA)],
                        r0.at[pl.ds(0, ROWS_EXTRA)])
        pltpu.sync_copy(r0.at[pl.ds(0, ROWS_EXTRA)],
                        out_hbm.at[c].at[pl.ds(w1, ROWS_EXTRA)])


_sc_segment_sum = functools.partial(
    pl.kernel,
    mesh=plsc.VectorSubcoreMesh(core_axis_name="c", subcore_axis_name="s"),
    out_type=jax.ShapeDtypeStruct((2, N_NODES, HALF), jnp.float32),
    scratch_types=[
        pltpu.VMEM((SECL * CHUNK,), jnp.int32),
        pltpu.VMEM((SECL * CHUNK,), jnp.int32),
        pltpu.VMEM((SECL * CHUNK,), jnp.int32),
        pltpu.VMEM((SECL * CHUNK,), jnp.int32),
        pltpu.VMEM((CHUNK,), jnp.int32),
        pltpu.VMEM((CHUNK,), jnp.int32),
        pltpu.VMEM((CHUNK,), jnp.int32),
        pltpu.VMEM((CHUNK, HALF), jnp.float32),
        pltpu.VMEM((CHUNK, HALF), jnp.float32),
        pltpu.VMEM((CHUNK, HALF), jnp.float32),
        pltpu.VMEM((TAIL,), jnp.int32),
        pltpu.VMEM((TAIL,), jnp.int32),
        pltpu.VMEM_SHARED((N_NODES, HALF), jnp.float32),
        pltpu.SemaphoreType.DMA,
        pltpu.SemaphoreType.DMA,
        pltpu.SemaphoreType.DMA,
        pltpu.SemaphoreType.DMA,
        pltpu.SemaphoreType.DMA,
        pltpu.SemaphoreType.DMA,
        pltpu.SemaphoreType.DMA,
        pltpu.SemaphoreType.DMA,
        pltpu.SemaphoreType.DMA,
    ],
)(_sc_seg_body)


def _mm_body(h_ref, w_ref, o_ref):
    o_ref[0] = jnp.dot(h_ref[...], w_ref[...],
                       preferred_element_type=jnp.float32)


def _matmul_split(h, w):
    bn = 2000
    return pl.pallas_call(
        _mm_body,
        grid=(N_NODES // bn, 2),
        in_specs=[
            pl.BlockSpec((bn, HID), lambda i, c: (i, 0)),
            pl.BlockSpec((HID, HALF), lambda i, c: (0, c)),
        ],
        out_specs=pl.BlockSpec((1, bn, HALF), lambda i, c: (c, i, 0)),
        out_shape=jax.ShapeDtypeStruct((2, N_NODES, HALF), jnp.float32),
    )(h, w)


def _gru_body(agg_ref, h_ref, wih_ref, whh_ref, bih_ref, bhh_ref, o_ref):
    agg = jnp.concatenate([agg_ref[0], agg_ref[1]], axis=1)
    h = h_ref[...]
    gi = lax.dot_general(agg, wih_ref[...], (((1,), (1,)), ((), ())),
                         preferred_element_type=jnp.float32) + bih_ref[...]
    gh = lax.dot_general(h, whh_ref[...], (((1,), (1,)), ((), ())),
                         preferred_element_type=jnp.float32) + bhh_ref[...]
    r = jax.nn.sigmoid(gi[:, :HID] + gh[:, :HID])
    z = jax.nn.sigmoid(gi[:, HID:2 * HID] + gh[:, HID:2 * HID])
    n = jnp.tanh(gi[:, 2 * HID:] + r * gh[:, 2 * HID:])
    o_ref[...] = (1.0 - z) * n + z * h


def _gru_mm_body(agg_ref, h_ref, wih_ref, whh_ref, bih_ref, bhh_ref,
                 wnext_ref, h_out, m_out):
    agg = jnp.concatenate([agg_ref[0], agg_ref[1]], axis=1)
    h = h_ref[...]
    gi = lax.dot_general(agg, wih_ref[...], (((1,), (1,)), ((), ())),
                         preferred_element_type=jnp.float32) + bih_ref[...]
    gh = lax.dot_general(h, whh_ref[...], (((1,), (1,)), ((), ())),
                         preferred_element_type=jnp.float32) + bhh_ref[...]
    r = jax.nn.sigmoid(gi[:, :HID] + gh[:, :HID])
    z = jax.nn.sigmoid(gi[:, HID:2 * HID] + gh[:, HID:2 * HID])
    n = jnp.tanh(gi[:, 2 * HID:] + r * gh[:, 2 * HID:])
    hn = (1.0 - z) * n + z * h
    h_out[...] = hn
    mm = jnp.dot(hn, wnext_ref[...], preferred_element_type=jnp.float32)
    m_out[0] = mm[:, :HALF]
    m_out[1] = mm[:, HALF:]


def _gru_mm(agg2, h, W_ih, W_hh, bih2, bhh2, wnext):
    bn = 2000
    return pl.pallas_call(
        _gru_mm_body,
        grid=(N_NODES // bn,),
        in_specs=[
            pl.BlockSpec((2, bn, HALF), lambda i: (0, i, 0)),
            pl.BlockSpec((bn, HID), lambda i: (i, 0)),
            pl.BlockSpec((3 * HID, HID), lambda i: (0, 0)),
            pl.BlockSpec((3 * HID, HID), lambda i: (0, 0)),
            pl.BlockSpec((1, 3 * HID), lambda i: (0, 0)),
            pl.BlockSpec((1, 3 * HID), lambda i: (0, 0)),
            pl.BlockSpec((HID, HID), lambda i: (0, 0)),
        ],
        out_specs=[
            pl.BlockSpec((bn, HID), lambda i: (i, 0)),
            pl.BlockSpec((2, bn, HALF), lambda i: (0, i, 0)),
        ],
        out_shape=[
            jax.ShapeDtypeStruct((N_NODES, HID), jnp.float32),
            jax.ShapeDtypeStruct((2, N_NODES, HALF), jnp.float32),
        ],
    )(agg2, h, W_ih, W_hh, bih2, bhh2, wnext)


def _gru(agg2, h, W_ih, W_hh, bih2, bhh2):
    bn = 2000
    return pl.pallas_call(
        _gru_body,
        grid=(N_NODES // bn,),
        in_specs=[
            pl.BlockSpec((2, bn, HALF), lambda i: (0, i, 0)),
            pl.BlockSpec((bn, HID), lambda i: (i, 0)),
            pl.BlockSpec((3 * HID, HID), lambda i: (0, 0)),
            pl.BlockSpec((3 * HID, HID), lambda i: (0, 0)),
            pl.BlockSpec((1, 3 * HID), lambda i: (0, 0)),
            pl.BlockSpec((1, 3 * HID), lambda i: (0, 0)),
        ],
        out_specs=pl.BlockSpec((bn, HID), lambda i: (i, 0)),
        out_shape=jax.ShapeDtypeStruct((N_NODES, HID), jnp.float32),
    )(agg2, h, W_ih, W_hh, bih2, bhh2)


def _pool_body(h_ref, b_ref, lw_ref, lb_ref, o_ref):
    bvec = b_ref[0, :]
    gids = lax.broadcasted_iota(jnp.int32, (NUM_GRAPHS, N_NODES), 0)
    onehot = (bvec[None, :] == gids).astype(jnp.float32)
    sums = jnp.dot(onehot, h_ref[...], preferred_element_type=jnp.float32)
    counts = jnp.sum(onehot, axis=1)
    pooled = sums / jnp.clip(counts, 1.0)[:, None]
    o_ref[...] = lax.dot_general(pooled, lw_ref[...], (((1,), (1,)), ((), ())),
                                 preferred_element_type=jnp.float32) + lb_ref[...]


def _pool_classify(h, batch2, lwp, lbp):
    return pl.pallas_call(
        _pool_body,
        out_shape=jax.ShapeDtypeStruct((NUM_GRAPHS, 128), jnp.float32),
    )(h, batch2, lwp, lbp)


def kernel(x, edge_index, batch, weight, W_ih, W_hh, b_ih, b_hh, lin_W, lin_b):
    src = edge_index[0].astype(jnp.int32)
    dst = edge_index[1].astype(jnp.int32)
    h = jnp.pad(x, ((0, 0), (0, HID - x.shape[1])))
    bih2 = b_ih[None, :]
    bhh2 = b_hh[None, :]
    m2 = _matmul_split(h, weight[0])
    for i in range(NUM_LAYERS - 1):
        agg2 = _sc_segment_sum(m2, src, dst)
        h, m2 = _gru_mm(agg2, h, W_ih, W_hh, bih2, bhh2, weight[i + 1])
    agg2 = _sc_segment_sum(m2, src, dst)
    h = _gru(agg2, h, W_ih, W_hh, bih2, bhh2)
    lwp = jnp.zeros((128, HID), jnp.float32).at[:NUM_CLASSES].set(lin_W)
    lbp = jnp.zeros((1, 128), jnp.float32).at[0, :NUM_CLASSES].set(lin_b)
    out = _pool_classify(h, batch.astype(jnp.int32)[None, :], lwp, lbp)
    return out[:, :NUM_CLASSES]
